# Initial kernel scaffold; baseline (speedup 1.0000x reference)
import jax
import jax.numpy as jnp
from jax.experimental import pallas as pl


def kernel(drug_x, drug_edge_index, drug_edge_attr, drug_batch, prot_x, prot_edge_index, prot_edge_attr, prot_batch, params):
    raise NotImplementedError("write your pallas kernel here")


# speedup vs baseline: 9.4153x; 9.4153x over previous
import functools

import numpy as np
import jax
import jax.numpy as jnp
from jax import lax
from jax.experimental import pallas as pl
from jax.experimental.pallas import tpu as pltpu
from jax.experimental.pallas import tpu_sc as plsc

H, HID, HD, NB, FD, ED = 4, 128, 32, 32, 128, 4
_BN_K = float(1.0 / np.sqrt(1.0 + 1e-5))
NW = 32
_SEL16 = np.zeros((128, 16), np.float32)
for _c in range(128):
    _SEL16[_c, _c // 32] = 1.0
_SEL16T = np.ascontiguousarray(_SEL16.T)


def _leaky(x, s):
    return jnp.where(x >= 0, x, s * x)


def _mm(x, w, b):
    M, K = x.shape
    Nout = w.shape[1]
    BM = 512 if M % 512 == 0 else M

    def body(x_ref, w_ref, b_ref, o_ref):
        o_ref[...] = (
            jnp.dot(x_ref[...], w_ref[...], preferred_element_type=jnp.float32)
            + b_ref[...]
        )

    return pl.pallas_call(
        body,
        grid=(M // BM,),
        in_specs=[
            pl.BlockSpec((BM, K), lambda i: (i, 0)),
            pl.BlockSpec((K, Nout), lambda i: (0, 0)),
            pl.BlockSpec((1, Nout), lambda i: (0, 0)),
        ],
        out_specs=pl.BlockSpec((BM, Nout), lambda i: (i, 0)),
        out_shape=jax.ShapeDtypeStruct((M, Nout), jnp.float32),
    )(x, w, b.reshape(1, -1))


def _edge_logits(gs, gd, ee, attrow, sel16):
    E2 = gs.shape[0]
    BE = 2048
    nb = E2 // BE

    def body(gs_ref, gd_ref, ee_ref, att_ref, sel_ref, w_ref, m_ref, macc):
        i = pl.program_id(0)
        v = _leaky(gs_ref[...] + gd_ref[...] + ee_ref[...], 0.2) * att_ref[...]
        lg = jnp.dot(v, sel_ref[...], preferred_element_type=jnp.float32)
        w_ref[...] = lg
        mb = jnp.max(lg[:, :4])

        @pl.when(i == 0)
        def _():
            macc[0, 0] = mb

        @pl.when(i > 0)
        def _():
            macc[0, 0] = jnp.maximum(macc[0, 0], mb)

        @pl.when(i == nb - 1)
        def _():
            m_ref[...] = jnp.full((1, 128), macc[0, 0], jnp.float32)

    return pl.pallas_call(
        body,
        grid=(nb,),
        in_specs=[
            pl.BlockSpec((BE, 128), lambda i: (i, 0)),
            pl.BlockSpec((BE, 128), lambda i: (i, 0)),
            pl.BlockSpec((BE, 128), lambda i: (i, 0)),
            pl.BlockSpec((1, 128), lambda i: (0, 0)),
            pl.BlockSpec((128, 16), lambda i: (0, 0)),
        ],
        out_specs=[
            pl.BlockSpec((BE, 16), lambda i: (i, 0)),
            pl.BlockSpec((1, 128), lambda i: (0, 0)),
        ],
        out_shape=[
            jax.ShapeDtypeStruct((E2, 16), jnp.float32),
            jax.ShapeDtypeStruct((1, 128), jnp.float32),
        ],
        scratch_shapes=[pltpu.SMEM((1, 1), jnp.float32)],
    )(gs, gd, ee, attrow, sel16)


def _loop_attr(accs, N):
    G, Np, D = accs.shape
    BN_ = 512

    def body(a_ref, o_ref):
        a = jnp.sum(a_ref[...], axis=0)
        deg = jnp.maximum(a[:, 4:5], 1.0)
        o_ref[...] = a / deg

    out = pl.pallas_call(
        body,
        grid=(N // BN_,),
        in_specs=[pl.BlockSpec((G, BN_, D), lambda i: (0, i, 0))],
        out_specs=pl.BlockSpec((BN_, D), lambda i: (i, 0)),
        out_shape=jax.ShapeDtypeStruct((N, D), jnp.float32),
    )(accs)
    return out[:, :ED]


def _edge_vu(gs, w16, m, sel16t):
    E2 = gs.shape[0]
    BE = 2048

    def body(gs_ref, w_ref, m_ref, sel_ref, o_ref):
        ew = jnp.exp(w_ref[...] - m_ref[0, 0])
        vu = gs_ref[...] * jnp.dot(ew, sel_ref[...],
                                   preferred_element_type=jnp.float32)
        o_ref[...] = jnp.concatenate(
            [vu, ew, jnp.zeros((BE, 112), jnp.float32)], axis=1)

    return pl.pallas_call(
        body,
        grid=(E2 // BE,),
        in_specs=[
            pl.BlockSpec((BE, 128), lambda i: (i, 0)),
            pl.BlockSpec((BE, 16), lambda i: (i, 0)),
            pl.BlockSpec((1, 128), lambda i: (0, 0)),
            pl.BlockSpec((16, 128), lambda i: (0, 0)),
        ],
        out_specs=pl.BlockSpec((BE, 256), lambda i: (i, 0)),
        out_shape=jax.ShapeDtypeStruct((E2, 256), jnp.float32),
    )(gs, w16, m, sel16t)


def _post(o2, consts, sel16t, N):
    BN_ = 512

    def body(o2_ref, c_ref, sel_ref, o_ref):
        agg = o2_ref[0] + o2_ref[1]
        num = agg[:, :128]
        den = jnp.dot(agg[:, 128:144], sel_ref[...],
                      preferred_element_type=jnp.float32)
        o = num / (den + 1e-16) + c_ref[0:1, :]
        y = o * (c_ref[1:2, :] * _BN_K) + c_ref[2:3, :]
        o_ref[...] = _leaky(y, 0.02)

    return pl.pallas_call(
        body,
        grid=(N // BN_,),
        in_specs=[
            pl.BlockSpec((2, BN_, 256), lambda i: (0, i, 0)),
            pl.BlockSpec((8, 128), lambda i: (0, 0)),
            pl.BlockSpec((16, 128), lambda i: (0, 0)),
        ],
        out_specs=pl.BlockSpec((BN_, 128), lambda i: (i, 0)),
        out_shape=jax.ShapeDtypeStruct((N, 128), jnp.float32),
    )(o2, consts, sel16t)


def _cross_attn(Q, KT, V, qb_b, kbT_b):
    QN = Q.shape[0]
    KN = V.shape[0]
    BQ = 256
    inv = float(1.0 / np.sqrt(HD))

    def body(q_ref, kt_ref, v_ref, qb_ref, kbt_ref, o_ref):
        iota_r = lax.broadcasted_iota(jnp.int32, (1, 128), 1)
        iota_c = lax.broadcasted_iota(jnp.int32, (128, 1), 0)
        qoh = (qb_ref[...] == iota_r).astype(jnp.float32)
        kohT = (kbt_ref[...] == iota_c).astype(jnp.float32)
        maskf = jnp.dot(qoh, kohT, preferred_element_type=jnp.float32)
        anyf = (jnp.sum(maskf, axis=1, keepdims=True) > 0.5).astype(jnp.float32)
        q = q_ref[...]
        kt = kt_ref[...]
        v = v_ref[...]
        outs = []
        for h in range(H):
            qh = q[:, h * HD:(h + 1) * HD]
            kth = kt[h * HD:(h + 1) * HD, :]
            vh = v[:, h * HD:(h + 1) * HD]
            s = jnp.dot(qh, kth, preferred_element_type=jnp.float32) * inv
            s = jnp.where(maskf > 0.5, s, -1e30)
            mx = jnp.max(s, axis=1, keepdims=True)
            p = jnp.exp(s - mx)
            ssum = jnp.sum(p, axis=1, keepdims=True)
            attn = p / ssum * anyf
            outs.append(jnp.dot(attn, vh, preferred_element_type=jnp.float32))
        o_ref[...] = jnp.concatenate(outs, axis=1)

    return pl.pallas_call(
        body,
        grid=(QN // BQ,),
        in_specs=[
            pl.BlockSpec((BQ, 128), lambda i: (i, 0)),
            pl.BlockSpec((128, KN), lambda i: (0, 0)),
            pl.BlockSpec((KN, 128), lambda i: (0, 0)),
            pl.BlockSpec((BQ, 128), lambda i: (i, 0)),
            pl.BlockSpec((128, KN), lambda i: (0, 0)),
        ],
        out_specs=pl.BlockSpec((BQ, 128), lambda i: (i, 0)),
        out_shape=jax.ShapeDtypeStruct((QN, 128), jnp.float32),
    )(Q, KT, V, qb_b, kbT_b)


def _fuse(x, a, fus):
    N = x.shape[0]
    BN_ = 512

    def body(x_ref, a_ref, f_ref, o_ref):
        f0 = f_ref[0, 0]
        f1 = f_ref[0, 1]
        mx = jnp.maximum(f0, f1)
        e0 = jnp.exp(f0 - mx)
        e1 = jnp.exp(f1 - mx)
        w0 = e0 / (e0 + e1)
        w1 = e1 / (e0 + e1)
        o_ref[...] = w0 * x_ref[...] + w1 * a_ref[...]

    return pl.pallas_call(
        body,
        grid=(N // BN_,),
        in_specs=[
            pl.BlockSpec((BN_, 128), lambda i: (i, 0)),
            pl.BlockSpec((BN_, 128), lambda i: (i, 0)),
            pl.BlockSpec((1, 2), lambda i: (0, 0)),
        ],
        out_specs=pl.BlockSpec((BN_, 128), lambda i: (i, 0)),
        out_shape=jax.ShapeDtypeStruct((N, 128), jnp.float32),
    )(x, a, fus)


def _pool(x, batch_b, batchT_b, w1, b1, w2b, b2b):
    N = x.shape[0]

    def body(x_ref, b_ref, bt_ref, w1_ref, b1_ref, w2_ref, b2_ref, o_ref):
        xx = x_ref[...]
        h1 = jnp.dot(xx, w1_ref[...], preferred_element_type=jnp.float32) + b1_ref[...]
        h1 = _leaky(h1, 0.02)
        g = jnp.dot(h1, w2_ref[...], preferred_element_type=jnp.float32) + b2_ref[0, 0]
        mx = jnp.max(g[:, :1])
        e = jnp.exp(g - mx)
        iota_r = lax.broadcasted_iota(jnp.int32, (1, 128), 1)
        iota_c = lax.broadcasted_iota(jnp.int32, (128, 1), 0)
        boh = (b_ref[...] == iota_r).astype(jnp.float32)
        bohT = (bt_ref[...] == iota_c).astype(jnp.float32)
        sseg = jnp.dot(bohT, e, preferred_element_type=jnp.float32)
        sden = jnp.dot(boh, sseg, preferred_element_type=jnp.float32)
        alpha = e / (sden + 1e-16)
        o_ref[...] = jnp.dot(bohT, xx * alpha, preferred_element_type=jnp.float32)

    return pl.pallas_call(
        body,
        out_shape=jax.ShapeDtypeStruct((128, 128), jnp.float32),
    )(x, batch_b, batchT_b, w1, b1, w2b, b2b)


def _head(c, o):
    def body(c_ref, w1_r, b1_r, g1_r, bb1_r, w2_r, b2_r, g2_r, bb2_r,
             w3_r, b3_r, g3_r, bb3_r, w4_r, b4_r, o_ref):
        def bnl(h, g, bb):
            return _leaky(h * (g[...] * _BN_K) + bb[...], 0.02)

        h = jnp.dot(c_ref[...], w1_r[...], preferred_element_type=jnp.float32) + b1_r[...]
        h = bnl(h, g1_r, bb1_r)
        h = jnp.dot(h, w2_r[...], preferred_element_type=jnp.float32) + b2_r[...]
        h = bnl(h, g2_r, bb2_r)
        h = jnp.dot(h, w3_r[...], preferred_element_type=jnp.float32) + b3_r[...]
        h = bnl(h, g3_r, bb3_r)
        o_ref[...] = jnp.dot(h, w4_r[...], preferred_element_type=jnp.float32) + b4_r[...]

    args = [
        c,
        o['l1w'], o['l1b'].reshape(1, -1), o['bn1']['g'].reshape(1, -1), o['bn1']['b'].reshape(1, -1),
        o['l2w'], o['l2b'].reshape(1, -1), o['bn2']['g'].reshape(1, -1), o['bn2']['b'].reshape(1, -1),
        o['l3w'], o['l3b'].reshape(1, -1), o['bn3']['g'].reshape(1, -1), o['bn3']['b'].reshape(1, -1),
        o['l4w'], o['l4b'].reshape(1, -1),
    ]
    return pl.pallas_call(
        body,
        out_shape=jax.ShapeDtypeStruct((NB, 1), jnp.float32),
    )(*args)


@functools.cache
def _sc_mesh():
    return plsc.VectorSubcoreMesh(core_axis_name="c", subcore_axis_name="s")


@functools.cache
def _sc_gather_fn(E, Nt, D):
    rpw = E // NW
    nch = rpw // 128

    @functools.partial(
        pl.kernel,
        out_type=jax.ShapeDtypeStruct((E, D), jnp.float32),
        mesh=_sc_mesh(),
        scratch_types=[
            pltpu.VMEM((128,), jnp.int32),
            pltpu.VMEM((128, D), jnp.float32),
            pltpu.SemaphoreType.DMA,
        ],
    )
    def k(table_hbm, idx_hbm, out_hbm, idx_v, rows_v, sem):
        wid = lax.axis_index("s") * 2 + lax.axis_index("c")
        base = wid * rpw
        for i in range(nch):
            off = base + i * 128
            pltpu.sync_copy(idx_hbm.at[pl.ds(off, 128)], idx_v)
            pltpu.async_copy(table_hbm.at[idx_v], rows_v, sem).wait()
            pltpu.sync_copy(rows_v, out_hbm.at[pl.ds(off, 128)])

    return k


def _sc_gather(table, idx):
    return _sc_gather_fn(idx.shape[0], table.shape[0], table.shape[1])(table, idx)


@functools.cache
def _sc_scatter_fn(E, Np, D):
    NS = D // 16
    ES = NW // NS
    C = 512
    rpe = E // ES
    nch = rpe // C
    assert E % (ES * C) == 0

    @functools.partial(
        pl.kernel,
        out_type=jax.ShapeDtypeStruct((ES, Np, D), jnp.float32),
        mesh=_sc_mesh(),
        scratch_types=[
            pltpu.VMEM((C,), jnp.int32),
            pltpu.VMEM((C, 16), jnp.float32),
            pltpu.VMEM((Np, 16), jnp.float32),
        ],
        compiler_params=pltpu.CompilerParams(
            needs_layout_passes=False, use_tc_tiling_on_sc=False),
    )
    def k(vals_hbm, idx_hbm, zeros_hbm, out_hbm, idx_v, buf, acc):
        cid = lax.axis_index("c")
        sid = lax.axis_index("s")
        wid = sid * 2 + cid
        sl = wid % NS
        eg = wid // NS
        pltpu.sync_copy(zeros_hbm, acc)
        base = eg * rpe
        iota16 = lax.iota(jnp.int32, 16)

        def chunk(i, carry):
            off = base + i * C
            pltpu.sync_copy(idx_hbm.at[pl.ds(off, C)], idx_v)
            pltpu.sync_copy(
                vals_hbm.at[pl.ds(off, C), pl.ds(sl * 16, 16)], buf)

            def group(j, c2):
                j16 = j * 16
                dst16 = plsc.load_gather(idx_v, [j16 + iota16])
                for l in range(16):
                    lvec = jnp.full((16,), l, jnp.int32)
                    v = plsc.load_gather(buf, [j16 + iota16, lvec])
                    plsc.addupdate_scatter(acc, [dst16, lvec], v)
                return c2

            return lax.fori_loop(0, C // 16, group, carry)

        lax.fori_loop(0, nch, chunk, 0)
        pltpu.sync_copy(acc, out_hbm.at[eg, :, pl.ds(sl * 16, 16)])

    return k


def _sc_scatter_add(vals, idx, Np):
    E, D = vals.shape
    zeros = jnp.zeros((Np, 16), jnp.float32)
    return _sc_scatter_fn(E, Np, D)(vals, idx, zeros)


def _encode_graph(x, ei, ea, enc, N, E):
    src, dst = ei[0], ei[1]
    Npad = N + 128
    ar = jnp.arange(N, dtype=jnp.int32)
    src2 = jnp.concatenate([src, ar])
    dst2 = jnp.concatenate([dst, ar])
    E2r = E + N
    E2 = ((E2r + 4095) // 4096) * 4096
    pad = E2 - E2r
    if pad:
        src2 = jnp.concatenate([src2, jnp.zeros((pad,), jnp.int32)])
        dst2_s = jnp.concatenate([dst2, jnp.full((pad,), N, jnp.int32)])
        dst2_g = jnp.concatenate([dst2, jnp.zeros((pad,), jnp.int32)])
    else:
        dst2_s = dst2
        dst2_g = dst2

    ea16 = jnp.concatenate(
        [ea, jnp.ones((E, 1), jnp.float32), jnp.zeros((E, 11), jnp.float32)], axis=1)
    accs = _sc_scatter_add(ea16, dst, Npad)
    loop = _loop_attr(accs, N)
    ea2 = jnp.concatenate([ea, loop], axis=0)
    if pad:
        ea2 = jnp.concatenate([ea2, jnp.zeros((pad, ED), jnp.float32)], axis=0)

    sel16 = jnp.asarray(_SEL16)
    sel16t = jnp.asarray(_SEL16T)
    zeros128 = jnp.zeros((HID,), jnp.float32)

    for lp, bp in zip(enc['layers'], enc['bns']):
        wcat = jnp.concatenate([lp['wl'], lp['wr']], axis=1)
        bcat = jnp.concatenate([lp['bl'], lp['br']])
        xlxr = _mm(x, wcat, bcat)
        xl = xlxr[:, :HID]
        xr = xlxr[:, HID:]
        ee = _mm(ea2, lp['we'], zeros128)
        gs = _sc_gather(xl, src2)
        gd = _sc_gather(xr, dst2_g)
        w16, m = _edge_logits(gs, gd, ee, lp['att'].reshape(1, HID), sel16)
        vu = _edge_vu(gs, w16, m, sel16t)
        o2 = _sc_scatter_add(vu, dst2_s, Npad)
        consts = jnp.concatenate(
            [lp['bias'][None], bp['g'][None], bp['b'][None],
             jnp.zeros((5, HID), jnp.float32)], axis=0)
        x = _post(o2, consts, sel16t, N)

    return _mm(x, enc['fw'], enc['fb'])


def _cross(qf, kf, qb_b, kbT_b, p):
    wq = jnp.transpose(p['wq'], (1, 0, 2)).reshape(HID, HID)
    wk = jnp.transpose(p['wk'], (1, 0, 2)).reshape(HID, HID)
    wv = jnp.transpose(p['wv'], (1, 0, 2)).reshape(HID, HID)
    q = _mm(qf, wq, p['bq'].reshape(-1))
    k = _mm(kf, wk, p['bk'].reshape(-1))
    v = _mm(kf, wv, p['bv'].reshape(-1))
    kt = jnp.transpose(k)
    ao = _cross_attn(q, kt, v, qb_b, kbT_b)
    return _mm(ao, p['wo'], p['bo'])


def kernel(drug_x, drug_edge_index, drug_edge_attr, drug_batch,
           prot_x, prot_edge_index, prot_edge_attr, prot_batch, params):
    DN = drug_x.shape[0]
    PN = prot_x.shape[0]
    DE = drug_edge_attr.shape[0]
    PE = prot_edge_attr.shape[0]

    x_d = _encode_graph(drug_x, drug_edge_index, drug_edge_attr,
                        params['drug_enc'], DN, DE)
    x_p = _encode_graph(prot_x, prot_edge_index, prot_edge_attr,
                        params['prot_enc'], PN, PE)

    db_b = jnp.broadcast_to(drug_batch[:, None], (DN, 128))
    pb_b = jnp.broadcast_to(prot_batch[:, None], (PN, 128))
    dbT_b = jnp.broadcast_to(drug_batch[None, :], (128, DN))
    pbT_b = jnp.broadcast_to(prot_batch[None, :], (128, PN))

    a1 = _cross(x_p, x_d, pb_b, dbT_b, params['x_d2p'])
    a2 = _cross(x_d, x_p, db_b, pbT_b, params['x_p2d'])
    fus = params['fusion'].reshape(1, 2)
    x_p = _fuse(x_p, a1, fus)
    x_d = _fuse(x_d, a2, fus)

    pw = params['pool']
    w2b = jnp.broadcast_to(pw['w2'], (HID // 2, 128))
    b2b = jnp.broadcast_to(pw['b2'].reshape(1, 1), (1, 128))
    b1r = pw['b1'].reshape(1, -1)
    xp = _pool(x_p, pb_b, pbT_b, pw['w1'], b1r, w2b, b2b)[:NB]
    xd = _pool(x_d, db_b, dbT_b, pw['w1'], b1r, w2b, b2b)[:NB]

    c = jnp.concatenate([xp, xd], axis=1)
    return _head(c, params['out'])


# speedup vs baseline: 13.1079x; 1.3922x over previous
import functools

import numpy as np
import jax
import jax.numpy as jnp
from jax import lax
from jax.experimental import pallas as pl
from jax.experimental.pallas import tpu as pltpu
from jax.experimental.pallas import tpu_sc as plsc

H, HID, HD, NB, FD, ED = 4, 128, 32, 32, 128, 4
_BN_K = float(1.0 / np.sqrt(1.0 + 1e-5))
NW = 32
_SEL16 = np.zeros((128, 16), np.float32)
for _c in range(128):
    _SEL16[_c, _c // 32] = 1.0
_SEL16T = np.ascontiguousarray(_SEL16.T)


def _leaky(x, s):
    return jnp.where(x >= 0, x, s * x)


def _mm(x, w, b):
    M, K = x.shape
    Nout = w.shape[1]
    BM = 512 if M % 512 == 0 else M

    def body(x_ref, w_ref, b_ref, o_ref):
        o_ref[...] = (
            jnp.dot(x_ref[...], w_ref[...], preferred_element_type=jnp.float32)
            + b_ref[...]
        )

    return pl.pallas_call(
        body,
        grid=(M // BM,),
        in_specs=[
            pl.BlockSpec((BM, K), lambda i: (i, 0)),
            pl.BlockSpec((K, Nout), lambda i: (0, 0)),
            pl.BlockSpec((1, Nout), lambda i: (0, 0)),
        ],
        out_specs=pl.BlockSpec((BM, Nout), lambda i: (i, 0)),
        out_shape=jax.ShapeDtypeStruct((M, Nout), jnp.float32),
    )(x, w, b.reshape(1, -1))


def _edge_logits(gs, gd, ee, attrow, sel16):
    E2 = gs.shape[0]
    BE = 2048
    nb = E2 // BE

    def body(gs_ref, gd_ref, ee_ref, att_ref, sel_ref, w_ref, m_ref, macc):
        i = pl.program_id(0)
        v = _leaky(gs_ref[...] + gd_ref[...] + ee_ref[...], 0.2) * att_ref[...]
        lg = jnp.dot(v, sel_ref[...], preferred_element_type=jnp.float32)
        w_ref[...] = lg
        mb = jnp.max(lg[:, :4])

        @pl.when(i == 0)
        def _():
            macc[0, 0] = mb

        @pl.when(i > 0)
        def _():
            macc[0, 0] = jnp.maximum(macc[0, 0], mb)

        @pl.when(i == nb - 1)
        def _():
            m_ref[...] = jnp.full((1, 128), macc[0, 0], jnp.float32)

    return pl.pallas_call(
        body,
        grid=(nb,),
        in_specs=[
            pl.BlockSpec((BE, 128), lambda i: (i, 0)),
            pl.BlockSpec((BE, 128), lambda i: (i, 0)),
            pl.BlockSpec((BE, 128), lambda i: (i, 0)),
            pl.BlockSpec((1, 128), lambda i: (0, 0)),
            pl.BlockSpec((128, 16), lambda i: (0, 0)),
        ],
        out_specs=[
            pl.BlockSpec((BE, 16), lambda i: (i, 0)),
            pl.BlockSpec((1, 128), lambda i: (0, 0)),
        ],
        out_shape=[
            jax.ShapeDtypeStruct((E2, 16), jnp.float32),
            jax.ShapeDtypeStruct((1, 128), jnp.float32),
        ],
        scratch_shapes=[pltpu.SMEM((1, 1), jnp.float32)],
    )(gs, gd, ee, attrow, sel16)


def _loop_attr(accs, N):
    G, Np, D = accs.shape
    BN_ = 512

    def body(a_ref, o_ref):
        a = jnp.sum(a_ref[...], axis=0)
        deg = jnp.maximum(a[:, 4:5], 1.0)
        o_ref[...] = a / deg

    out = pl.pallas_call(
        body,
        grid=(N // BN_,),
        in_specs=[pl.BlockSpec((G, BN_, D), lambda i: (0, i, 0))],
        out_specs=pl.BlockSpec((BN_, D), lambda i: (i, 0)),
        out_shape=jax.ShapeDtypeStruct((N, D), jnp.float32),
    )(accs)
    return out[:, :ED]


def _edge_vu(gs, w16, m, sel16t):
    E2 = gs.shape[0]
    BE = 2048

    def body(gs_ref, w_ref, m_ref, sel_ref, o_ref):
        ew = jnp.exp(w_ref[...] - m_ref[0, 0])
        vu = gs_ref[...] * jnp.dot(ew, sel_ref[...],
                                   preferred_element_type=jnp.float32)
        o_ref[...] = jnp.concatenate(
            [vu, ew, jnp.zeros((BE, 112), jnp.float32)], axis=1)

    return pl.pallas_call(
        body,
        grid=(E2 // BE,),
        in_specs=[
            pl.BlockSpec((BE, 128), lambda i: (i, 0)),
            pl.BlockSpec((BE, 16), lambda i: (i, 0)),
            pl.BlockSpec((1, 128), lambda i: (0, 0)),
            pl.BlockSpec((16, 128), lambda i: (0, 0)),
        ],
        out_specs=pl.BlockSpec((BE, 256), lambda i: (i, 0)),
        out_shape=jax.ShapeDtypeStruct((E2, 256), jnp.float32),
    )(gs, w16, m, sel16t)


def _post(o2, consts, sel16t, N):
    BN_ = 512

    def body(o2_ref, c_ref, sel_ref, o_ref):
        agg = o2_ref[0] + o2_ref[1]
        num = agg[:, :128]
        den = jnp.dot(agg[:, 128:144], sel_ref[...],
                      preferred_element_type=jnp.float32)
        o = num / (den + 1e-16) + c_ref[0:1, :]
        y = o * (c_ref[1:2, :] * _BN_K) + c_ref[2:3, :]
        o_ref[...] = _leaky(y, 0.02)

    return pl.pallas_call(
        body,
        grid=(N // BN_,),
        in_specs=[
            pl.BlockSpec((2, BN_, 256), lambda i: (0, i, 0)),
            pl.BlockSpec((8, 128), lambda i: (0, 0)),
            pl.BlockSpec((16, 128), lambda i: (0, 0)),
        ],
        out_specs=pl.BlockSpec((BN_, 128), lambda i: (i, 0)),
        out_shape=jax.ShapeDtypeStruct((N, 128), jnp.float32),
    )(o2, consts, sel16t)


def _cross_attn(Q, KT, V, qb_b, kbT_b):
    QN = Q.shape[0]
    KN = V.shape[0]
    BQ = 256
    inv = float(1.0 / np.sqrt(HD))

    def body(q_ref, kt_ref, v_ref, qb_ref, kbt_ref, o_ref):
        iota_r = lax.broadcasted_iota(jnp.int32, (1, 128), 1)
        iota_c = lax.broadcasted_iota(jnp.int32, (128, 1), 0)
        qoh = (qb_ref[...] == iota_r).astype(jnp.float32)
        kohT = (kbt_ref[...] == iota_c).astype(jnp.float32)
        maskf = jnp.dot(qoh, kohT, preferred_element_type=jnp.float32)
        anyf = (jnp.sum(maskf, axis=1, keepdims=True) > 0.5).astype(jnp.float32)
        q = q_ref[...]
        kt = kt_ref[...]
        v = v_ref[...]
        outs = []
        for h in range(H):
            qh = q[:, h * HD:(h + 1) * HD]
            kth = kt[h * HD:(h + 1) * HD, :]
            vh = v[:, h * HD:(h + 1) * HD]
            s = jnp.dot(qh, kth, preferred_element_type=jnp.float32) * inv
            s = jnp.where(maskf > 0.5, s, -1e30)
            mx = jnp.max(s, axis=1, keepdims=True)
            p = jnp.exp(s - mx)
            ssum = jnp.sum(p, axis=1, keepdims=True)
            attn = p / ssum * anyf
            outs.append(jnp.dot(attn, vh, preferred_element_type=jnp.float32))
        o_ref[...] = jnp.concatenate(outs, axis=1)

    return pl.pallas_call(
        body,
        grid=(QN // BQ,),
        in_specs=[
            pl.BlockSpec((BQ, 128), lambda i: (i, 0)),
            pl.BlockSpec((128, KN), lambda i: (0, 0)),
            pl.BlockSpec((KN, 128), lambda i: (0, 0)),
            pl.BlockSpec((BQ, 128), lambda i: (i, 0)),
            pl.BlockSpec((128, KN), lambda i: (0, 0)),
        ],
        out_specs=pl.BlockSpec((BQ, 128), lambda i: (i, 0)),
        out_shape=jax.ShapeDtypeStruct((QN, 128), jnp.float32),
    )(Q, KT, V, qb_b, kbT_b)


def _fuse(x, a, fus):
    N = x.shape[0]
    BN_ = 512

    def body(x_ref, a_ref, f_ref, o_ref):
        f0 = f_ref[0, 0]
        f1 = f_ref[0, 1]
        mx = jnp.maximum(f0, f1)
        e0 = jnp.exp(f0 - mx)
        e1 = jnp.exp(f1 - mx)
        w0 = e0 / (e0 + e1)
        w1 = e1 / (e0 + e1)
        o_ref[...] = w0 * x_ref[...] + w1 * a_ref[...]

    return pl.pallas_call(
        body,
        grid=(N // BN_,),
        in_specs=[
            pl.BlockSpec((BN_, 128), lambda i: (i, 0)),
            pl.BlockSpec((BN_, 128), lambda i: (i, 0)),
            pl.BlockSpec((1, 2), lambda i: (0, 0)),
        ],
        out_specs=pl.BlockSpec((BN_, 128), lambda i: (i, 0)),
        out_shape=jax.ShapeDtypeStruct((N, 128), jnp.float32),
    )(x, a, fus)


def _pool(x, batch_b, batchT_b, w1, b1, w2b, b2b):
    N = x.shape[0]

    def body(x_ref, b_ref, bt_ref, w1_ref, b1_ref, w2_ref, b2_ref, o_ref):
        xx = x_ref[...]
        h1 = jnp.dot(xx, w1_ref[...], preferred_element_type=jnp.float32) + b1_ref[...]
        h1 = _leaky(h1, 0.02)
        g = jnp.dot(h1, w2_ref[...], preferred_element_type=jnp.float32) + b2_ref[0, 0]
        mx = jnp.max(g[:, :1])
        e = jnp.exp(g - mx)
        iota_r = lax.broadcasted_iota(jnp.int32, (1, 128), 1)
        iota_c = lax.broadcasted_iota(jnp.int32, (128, 1), 0)
        boh = (b_ref[...] == iota_r).astype(jnp.float32)
        bohT = (bt_ref[...] == iota_c).astype(jnp.float32)
        sseg = jnp.dot(bohT, e, preferred_element_type=jnp.float32)
        sden = jnp.dot(boh, sseg, preferred_element_type=jnp.float32)
        alpha = e / (sden + 1e-16)
        o_ref[...] = jnp.dot(bohT, xx * alpha, preferred_element_type=jnp.float32)

    return pl.pallas_call(
        body,
        out_shape=jax.ShapeDtypeStruct((128, 128), jnp.float32),
    )(x, batch_b, batchT_b, w1, b1, w2b, b2b)


def _head(c, o):
    def body(c_ref, w1_r, b1_r, g1_r, bb1_r, w2_r, b2_r, g2_r, bb2_r,
             w3_r, b3_r, g3_r, bb3_r, w4_r, b4_r, o_ref):
        def bnl(h, g, bb):
            return _leaky(h * (g[...] * _BN_K) + bb[...], 0.02)

        h = jnp.dot(c_ref[...], w1_r[...], preferred_element_type=jnp.float32) + b1_r[...]
        h = bnl(h, g1_r, bb1_r)
        h = jnp.dot(h, w2_r[...], preferred_element_type=jnp.float32) + b2_r[...]
        h = bnl(h, g2_r, bb2_r)
        h = jnp.dot(h, w3_r[...], preferred_element_type=jnp.float32) + b3_r[...]
        h = bnl(h, g3_r, bb3_r)
        o_ref[...] = jnp.dot(h, w4_r[...], preferred_element_type=jnp.float32) + b4_r[...]

    args = [
        c,
        o['l1w'], o['l1b'].reshape(1, -1), o['bn1']['g'].reshape(1, -1), o['bn1']['b'].reshape(1, -1),
        o['l2w'], o['l2b'].reshape(1, -1), o['bn2']['g'].reshape(1, -1), o['bn2']['b'].reshape(1, -1),
        o['l3w'], o['l3b'].reshape(1, -1), o['bn3']['g'].reshape(1, -1), o['bn3']['b'].reshape(1, -1),
        o['l4w'], o['l4b'].reshape(1, -1),
    ]
    return pl.pallas_call(
        body,
        out_shape=jax.ShapeDtypeStruct((NB, 1), jnp.float32),
    )(*args)


@functools.cache
def _sc_mesh():
    return plsc.VectorSubcoreMesh(core_axis_name="c", subcore_axis_name="s")


@functools.cache
def _sc_gather_fn(E, Nt, D):
    rpw = E // NW
    nch = rpw // 128

    @functools.partial(
        pl.kernel,
        out_type=jax.ShapeDtypeStruct((E, D), jnp.float32),
        mesh=_sc_mesh(),
        scratch_types=[
            pltpu.VMEM((128,), jnp.int32),
            pltpu.VMEM((128, D), jnp.float32),
            pltpu.SemaphoreType.DMA,
        ],
    )
    def k(table_hbm, idx_hbm, out_hbm, idx_v, rows_v, sem):
        wid = lax.axis_index("s") * 2 + lax.axis_index("c")
        base = wid * rpw
        for i in range(nch):
            off = base + i * 128
            pltpu.sync_copy(idx_hbm.at[pl.ds(off, 128)], idx_v)
            pltpu.async_copy(table_hbm.at[idx_v], rows_v, sem).wait()
            pltpu.sync_copy(rows_v, out_hbm.at[pl.ds(off, 128)])

    return k


def _sc_gather(table, idx):
    return _sc_gather_fn(idx.shape[0], table.shape[0], table.shape[1])(table, idx)


@functools.cache
def _sc_scatter_fn(E, Np, D):
    NS = D // 16
    ES = NW // NS
    C = 512
    rpe = E // ES
    nch = rpe // C
    assert E % (ES * C) == 0

    @functools.partial(
        pl.kernel,
        out_type=jax.ShapeDtypeStruct((ES, Np, D), jnp.float32),
        mesh=_sc_mesh(),
        scratch_types=[
            pltpu.VMEM((C,), jnp.int32),
            pltpu.VMEM((C, 16), jnp.float32),
            pltpu.VMEM((Np, 16), jnp.float32),
        ],
        compiler_params=pltpu.CompilerParams(
            needs_layout_passes=False, use_tc_tiling_on_sc=False),
    )
    def k(vals_hbm, idx_hbm, zeros_hbm, out_hbm, idx_v, buf, acc):
        cid = lax.axis_index("c")
        sid = lax.axis_index("s")
        wid = sid * 2 + cid
        sl = wid % NS
        eg = wid // NS
        pltpu.sync_copy(zeros_hbm, acc)
        base = eg * rpe
        iota16 = lax.iota(jnp.int32, 16)
        colvecs = [(iota16 + o) & 15 for o in range(16)]

        def chunk(i, carry):
            off = base + i * C
            pltpu.sync_copy(idx_hbm.at[pl.ds(off, C)], idx_v)
            pltpu.sync_copy(
                vals_hbm.at[pl.ds(off, C), pl.ds(sl * 16, 16)], buf)

            def group(j, c2):
                rows = j * 16 + iota16
                dst16 = plsc.load_gather(idx_v, [rows])
                for o in range(16):
                    col = colvecs[o]
                    v = plsc.load_gather(buf, [rows, col])
                    plsc.addupdate_scatter(acc, [dst16, col], v)
                return c2

            return lax.fori_loop(0, C // 16, group, carry)

        lax.fori_loop(0, nch, chunk, 0)
        pltpu.sync_copy(acc, out_hbm.at[eg, :, pl.ds(sl * 16, 16)])

    return k


def _sc_scatter_add(vals, idx, Np):
    E, D = vals.shape
    zeros = jnp.zeros((Np, 16), jnp.float32)
    return _sc_scatter_fn(E, Np, D)(vals, idx, zeros)


def _encode_graph(x, ei, ea, enc, N, E):
    src, dst = ei[0], ei[1]
    Npad = N + 128
    ar = jnp.arange(N, dtype=jnp.int32)
    src2 = jnp.concatenate([src, ar])
    dst2 = jnp.concatenate([dst, ar])
    E2r = E + N
    E2 = ((E2r + 4095) // 4096) * 4096
    pad = E2 - E2r
    if pad:
        src2 = jnp.concatenate([src2, jnp.zeros((pad,), jnp.int32)])
        dst2_s = jnp.concatenate([dst2, jnp.full((pad,), N, jnp.int32)])
        dst2_g = jnp.concatenate([dst2, jnp.zeros((pad,), jnp.int32)])
    else:
        dst2_s = dst2
        dst2_g = dst2

    ea16 = jnp.concatenate(
        [ea, jnp.ones((E, 1), jnp.float32), jnp.zeros((E, 11), jnp.float32)], axis=1)
    accs = _sc_scatter_add(ea16, dst, Npad)
    loop = _loop_attr(accs, N)
    ea2 = jnp.concatenate([ea, loop], axis=0)
    if pad:
        ea2 = jnp.concatenate([ea2, jnp.zeros((pad, ED), jnp.float32)], axis=0)

    sel16 = jnp.asarray(_SEL16)
    sel16t = jnp.asarray(_SEL16T)
    zeros128 = jnp.zeros((HID,), jnp.float32)

    for lp, bp in zip(enc['layers'], enc['bns']):
        wcat = jnp.concatenate([lp['wl'], lp['wr']], axis=1)
        bcat = jnp.concatenate([lp['bl'], lp['br']])
        xlxr = _mm(x, wcat, bcat)
        xl = xlxr[:, :HID]
        xr = xlxr[:, HID:]
        ee = _mm(ea2, lp['we'], zeros128)
        gs = _sc_gather(xl, src2)
        gd = _sc_gather(xr, dst2_g)
        w16, m = _edge_logits(gs, gd, ee, lp['att'].reshape(1, HID), sel16)
        vu = _edge_vu(gs, w16, m, sel16t)
        o2 = _sc_scatter_add(vu, dst2_s, Npad)
        consts = jnp.concatenate(
            [lp['bias'][None], bp['g'][None], bp['b'][None],
             jnp.zeros((5, HID), jnp.float32)], axis=0)
        x = _post(o2, consts, sel16t, N)

    return _mm(x, enc['fw'], enc['fb'])


def _cross(qf, kf, qb_b, kbT_b, p):
    wq = jnp.transpose(p['wq'], (1, 0, 2)).reshape(HID, HID)
    wk = jnp.transpose(p['wk'], (1, 0, 2)).reshape(HID, HID)
    wv = jnp.transpose(p['wv'], (1, 0, 2)).reshape(HID, HID)
    q = _mm(qf, wq, p['bq'].reshape(-1))
    k = _mm(kf, wk, p['bk'].reshape(-1))
    v = _mm(kf, wv, p['bv'].reshape(-1))
    kt = jnp.transpose(k)
    ao = _cross_attn(q, kt, v, qb_b, kbT_b)
    return _mm(ao, p['wo'], p['bo'])


def kernel(drug_x, drug_edge_index, drug_edge_attr, drug_batch,
           prot_x, prot_edge_index, prot_edge_attr, prot_batch, params):
    DN = drug_x.shape[0]
    PN = prot_x.shape[0]
    DE = drug_edge_attr.shape[0]
    PE = prot_edge_attr.shape[0]

    x_d = _encode_graph(drug_x, drug_edge_index, drug_edge_attr,
                        params['drug_enc'], DN, DE)
    x_p = _encode_graph(prot_x, prot_edge_index, prot_edge_attr,
                        params['prot_enc'], PN, PE)

    db_b = jnp.broadcast_to(drug_batch[:, None], (DN, 128))
    pb_b = jnp.broadcast_to(prot_batch[:, None], (PN, 128))
    dbT_b = jnp.broadcast_to(drug_batch[None, :], (128, DN))
    pbT_b = jnp.broadcast_to(prot_batch[None, :], (128, PN))

    a1 = _cross(x_p, x_d, pb_b, dbT_b, params['x_d2p'])
    a2 = _cross(x_d, x_p, db_b, pbT_b, params['x_p2d'])
    fus = params['fusion'].reshape(1, 2)
    x_p = _fuse(x_p, a1, fus)
    x_d = _fuse(x_d, a2, fus)

    pw = params['pool']
    w2b = jnp.broadcast_to(pw['w2'], (HID // 2, 128))
    b2b = jnp.broadcast_to(pw['b2'].reshape(1, 1), (1, 128))
    b1r = pw['b1'].reshape(1, -1)
    xp = _pool(x_p, pb_b, pbT_b, pw['w1'], b1r, w2b, b2b)[:NB]
    xd = _pool(x_d, db_b, dbT_b, pw['w1'], b1r, w2b, b2b)[:NB]

    c = jnp.concatenate([xp, xd], axis=1)
    return _head(c, params['out'])
